# Initial kernel scaffold; baseline (speedup 1.0000x reference)
import jax
import jax.numpy as jnp
from jax.experimental import pallas as pl


def kernel(x, edge_index, edge_label_index, W1, b1, W2, b2, W3, b3, g1, be1, g2, be2, Wp1, bp1, Wp2, bp2, Wp3, bp3):
    raise NotImplementedError("write your pallas kernel here")


# speedup vs baseline: 2.4841x; 2.4841x over previous
import functools

import jax
import jax.numpy as jnp
from jax import lax
from jax.experimental import pallas as pl
from jax.experimental.pallas import tpu as pltpu
from jax.experimental.pallas import tpu_sc as plsc

N = 10000
NP = 10240
E = 160000
EP = 161792
EL = 160000
ELP = 163840
DIN = 256
H = 512
NC = 2
NS = 16
EB = 128
R = 1024
R2 = 2048

@functools.lru_cache(maxsize=None)
def _get_mesh():
    return plsc.VectorSubcoreMesh(
        core_axis_name="c", subcore_axis_name="s",
        num_cores=NC, num_subcores=NS)


def _sc_deg_body(dst_hbm, ones_hbm, out_hbm, idx_v, ones_v, acc, sem):
    cid = lax.axis_index("c")
    sid = lax.axis_index("s")

    @pl.when(cid == 0)
    def _():
        slab = NP // NS
        r0 = sid * slab
        pltpu.sync_copy(ones_hbm.at[pl.ds(r0, slab)], acc.at[pl.ds(r0, slab)])
        pltpu.sync_copy(ones_hbm.at[pl.ds(0, EB)], ones_v)
        plsc.subcore_barrier()
        epw = EP // NS
        nblk = epw // EB

        def blk(j, carry):
            off = sid * epw + j * EB
            pltpu.sync_copy(dst_hbm.at[pl.ds(off, EB)], idx_v)
            pltpu.sync_copy(ones_v, acc.at[idx_v], add=True)
            return carry

        lax.fori_loop(0, nblk, blk, 0)
        plsc.subcore_barrier()
        pltpu.sync_copy(acc.at[pl.ds(r0, slab)], out_hbm.at[pl.ds(r0, slab)])


@functools.partial(jax.jit)
def _sc_deg(dst_p, ones16):
    return pl.kernel(
        _sc_deg_body,
        out_type=jax.ShapeDtypeStruct((NP, 16), jnp.float32),
        mesh=_get_mesh(),
        scratch_types=[
            pltpu.VMEM((EB,), jnp.int32),
            pltpu.VMEM((EB, 16), jnp.float32),
            pltpu.VMEM_SHARED((NP, 16), jnp.float32),
            pltpu.SemaphoreType.DMA,
        ],
    )(dst_p, ones16)


def _sc_seg_body(t0, t1, t2, t3, src_hbm, dst_hbm, o0, o1, o2, o3,
                 idx_s, idx_d, rows_v, acc, sem):
    cid = lax.axis_index("c")
    sid = lax.axis_index("s")
    slab = NP // NS
    r0 = sid * slab
    epw = EP // NS
    nblk = epw // EB

    def do_chunk(t_hbm, o_hbm):
        pltpu.sync_copy(t_hbm.at[pl.ds(r0, slab)], acc.at[pl.ds(r0, slab)])
        plsc.subcore_barrier()

        def blk(j, carry):
            off = sid * epw + j * EB
            pltpu.sync_copy(src_hbm.at[pl.ds(off, EB)], idx_s)
            pltpu.sync_copy(dst_hbm.at[pl.ds(off, EB)], idx_d)
            pltpu.async_copy(t_hbm.at[idx_s], rows_v, sem).wait()
            pltpu.sync_copy(rows_v, acc.at[idx_d], add=True)
            return carry

        lax.fori_loop(0, nblk, blk, 0)
        plsc.subcore_barrier()
        pltpu.sync_copy(acc.at[pl.ds(r0, slab)], o_hbm.at[pl.ds(r0, slab)])
        plsc.subcore_barrier()

    @pl.when(cid == 0)
    def _():
        do_chunk(t0, o0)
        do_chunk(t1, o1)

    @pl.when(cid == 1)
    def _():
        do_chunk(t2, o2)
        do_chunk(t3, o3)


@functools.partial(jax.jit)
def _sc_seg(tc, src_p, dst_p):
    chunk = jax.ShapeDtypeStruct((NP, 128), jnp.float32)
    outs = pl.kernel(
        _sc_seg_body,
        out_type=(chunk,) * 4,
        mesh=_get_mesh(),
        scratch_types=[
            pltpu.VMEM((EB,), jnp.int32),
            pltpu.VMEM((EB,), jnp.int32),
            pltpu.VMEM((EB, 128), jnp.float32),
            pltpu.VMEM_SHARED((NP, 128), jnp.float32),
            pltpu.SemaphoreType.DMA,
        ],
    )(tc[0], tc[1], tc[2], tc[3], src_p, dst_p)
    return outs


def _sc_pair_body(p10, p11, p12, p13, p20, p21, p22, p23,
                  a_hbm, b_hbm, bp1_hbm, g0, g1, g2, g3,
                  idx_a, idx_b, rows_a, rows_b, bias_v, sem):
    cid = lax.axis_index("c")
    sid = lax.axis_index("s")
    wid = sid * NC + cid
    rpw = ELP // (NC * NS)
    nblk = rpw // EB
    pltpu.sync_copy(bp1_hbm, bias_v)
    tabs = ((p10, p20, g0, 0), (p11, p21, g1, 1),
            (p12, p22, g2, 2), (p13, p23, g3, 3))

    def blk(j, carry):
        roff = wid * rpw + j * EB
        pltpu.sync_copy(a_hbm.at[pl.ds(roff, EB)], idx_a)
        pltpu.sync_copy(b_hbm.at[pl.ds(roff, EB)], idx_b)
        for pa, pb, go, c in tabs:
            pltpu.async_copy(pa.at[idx_a], rows_a, sem).wait()
            pltpu.async_copy(pb.at[idx_b], rows_b, sem).wait()

            def row(r, cc):
                for jj in range(8):
                    sl = pl.ds(jj * 16, 16)
                    va = rows_a[r, sl]
                    vb = rows_b[r, sl]
                    bb = bias_v[pl.ds(c * 128 + jj * 16, 16)]
                    rows_a[r, sl] = jnp.maximum(va + vb + bb, 0.0)
                return cc

            lax.fori_loop(0, EB, row, 0)
            pltpu.sync_copy(rows_a, go.at[pl.ds(roff, EB)])
        return carry

    lax.fori_loop(0, nblk, blk, 0)


@functools.partial(jax.jit)
def _sc_pair(p1c, p2c, a_p, b_p, bp1):
    chunk = jax.ShapeDtypeStruct((ELP, 128), jnp.float32)
    return pl.kernel(
        _sc_pair_body,
        out_type=(chunk,) * 4,
        mesh=_get_mesh(),
        scratch_types=[
            pltpu.VMEM((EB,), jnp.int32),
            pltpu.VMEM((EB,), jnp.int32),
            pltpu.VMEM((EB, 128), jnp.float32),
            pltpu.VMEM((EB, 128), jnp.float32),
            pltpu.VMEM((H,), jnp.float32),
            pltpu.SemaphoreType.DMA,
        ],
    )(p1c[0], p1c[1], p1c[2], p1c[3], p2c[0], p2c[1], p2c[2], p2c[3],
      a_p, b_p, bp1)


def _dinv_block(deg_blk, i, rows):
    d = deg_blk[:, 0:1]
    ri = lax.broadcasted_iota(jnp.int32, (rows, 1), 0) + i * rows
    return jnp.where(ri < N, lax.rsqrt(d), 0.0)


def _tc_proj_body(x_ref, w_ref, deg_ref, o0, o1, o2, o3):
    i = pl.program_id(0)
    m = jnp.dot(x_ref[...], w_ref[...], preferred_element_type=jnp.float32)
    t = m * _dinv_block(deg_ref[...], i, R)
    o0[...] = t[:, 0:128]
    o1[...] = t[:, 128:256]
    o2[...] = t[:, 256:384]
    o3[...] = t[:, 384:512]


@functools.partial(jax.jit)
def _tc_proj(x_pad, W1, deg):
    chunk = jax.ShapeDtypeStruct((NP, 128), jnp.float32)
    grid = NP // R
    return pl.pallas_call(
        _tc_proj_body,
        grid=(grid,),
        in_specs=[
            pl.BlockSpec((R, DIN), lambda i: (i, 0)),
            pl.BlockSpec((DIN, H), lambda i: (0, 0)),
            pl.BlockSpec((R, 16), lambda i: (i, 0)),
        ],
        out_specs=[pl.BlockSpec((R, 128), lambda i: (i, 0))] * 4,
        out_shape=[chunk] * 4,
    )(x_pad, W1, deg)


def _y_block(s_refs, deg_ref, b_ref, i):
    s = jnp.concatenate([r[...] for r in s_refs], axis=1)
    dinv = _dinv_block(deg_ref[...], i, R)
    return s * dinv + b_ref[...], dinv


def _tc_stats_body(s0, s1, s2, s3, deg_ref, b_ref, out_ref):
    i = pl.program_id(0)
    y, _ = _y_block((s0, s1, s2, s3), deg_ref, b_ref, i)
    ri = lax.broadcasted_iota(jnp.int32, (R, 1), 0) + i * R
    ym = jnp.where(ri < N, y, 0.0)
    su = jnp.sum(ym, axis=0, keepdims=True)
    sq = jnp.sum(ym * ym, axis=0, keepdims=True)
    upd = jnp.concatenate([su, sq, jnp.zeros((6, H), jnp.float32)], axis=0)

    @pl.when(i == 0)
    def _():
        out_ref[...] = jnp.zeros((8, H), jnp.float32)

    out_ref[...] += upd


@functools.partial(jax.jit)
def _tc_stats(sc, deg, b_row):
    grid = NP // R
    return pl.pallas_call(
        _tc_stats_body,
        grid=(grid,),
        in_specs=[pl.BlockSpec((R, 128), lambda i: (i, 0))] * 4 + [
            pl.BlockSpec((R, 16), lambda i: (i, 0)),
            pl.BlockSpec((1, H), lambda i: (0, 0)),
        ],
        out_specs=pl.BlockSpec((8, H), lambda i: (0, 0)),
        out_shape=jax.ShapeDtypeStruct((8, H), jnp.float32),
    )(sc[0], sc[1], sc[2], sc[3], deg, b_row)


def _tc_layer_body(s0, s1, s2, s3, deg_ref, st_ref, b_ref, g_ref, be_ref,
                   w_ref, o0, o1, o2, o3):
    i = pl.program_id(0)
    y, dinv = _y_block((s0, s1, s2, s3), deg_ref, b_ref, i)
    st = st_ref[...]
    mu = st[0:1, :] * (1.0 / N)
    var = st[1:2, :] * (1.0 / N) - mu * mu
    hn = (y - mu) * lax.rsqrt(var + 1e-5) * g_ref[...] + be_ref[...]
    h = jnp.maximum(hn, 0.0)
    t = jnp.dot(h, w_ref[...], preferred_element_type=jnp.float32) * dinv
    o0[...] = t[:, 0:128]
    o1[...] = t[:, 128:256]
    o2[...] = t[:, 256:384]
    o3[...] = t[:, 384:512]


@functools.partial(jax.jit)
def _tc_layer(sc, deg, st, b_row, g_row, be_row, Wn):
    chunk = jax.ShapeDtypeStruct((NP, 128), jnp.float32)
    grid = NP // R
    return pl.pallas_call(
        _tc_layer_body,
        grid=(grid,),
        in_specs=[pl.BlockSpec((R, 128), lambda i: (i, 0))] * 4 + [
            pl.BlockSpec((R, 16), lambda i: (i, 0)),
            pl.BlockSpec((8, H), lambda i: (0, 0)),
            pl.BlockSpec((1, H), lambda i: (0, 0)),
            pl.BlockSpec((1, H), lambda i: (0, 0)),
            pl.BlockSpec((1, H), lambda i: (0, 0)),
            pl.BlockSpec((H, H), lambda i: (0, 0)),
        ],
        out_specs=[pl.BlockSpec((R, 128), lambda i: (i, 0))] * 4,
        out_shape=[chunk] * 4,
    )(sc[0], sc[1], sc[2], sc[3], deg, st, b_row, g_row, be_row, Wn)


def _tc_zproj_body(s0, s1, s2, s3, deg_ref, b_ref, wa_ref, wb_ref,
                   o10, o11, o12, o13, o20, o21, o22, o23):
    i = pl.program_id(0)
    z, _ = _y_block((s0, s1, s2, s3), deg_ref, b_ref, i)
    p1 = jnp.dot(z, wa_ref[...], preferred_element_type=jnp.float32)
    p2 = jnp.dot(z, wb_ref[...], preferred_element_type=jnp.float32)
    o10[...] = p1[:, 0:128]
    o11[...] = p1[:, 128:256]
    o12[...] = p1[:, 256:384]
    o13[...] = p1[:, 384:512]
    o20[...] = p2[:, 0:128]
    o21[...] = p2[:, 128:256]
    o22[...] = p2[:, 256:384]
    o23[...] = p2[:, 384:512]


@functools.partial(jax.jit)
def _tc_zproj(sc, deg, b3_row, Wa, Wb):
    chunk = jax.ShapeDtypeStruct((NP, 128), jnp.float32)
    grid = NP // R
    outs = pl.pallas_call(
        _tc_zproj_body,
        grid=(grid,),
        in_specs=[pl.BlockSpec((R, 128), lambda i: (i, 0))] * 4 + [
            pl.BlockSpec((R, 16), lambda i: (i, 0)),
            pl.BlockSpec((1, H), lambda i: (0, 0)),
            pl.BlockSpec((H, H), lambda i: (0, 0)),
            pl.BlockSpec((H, H), lambda i: (0, 0)),
        ],
        out_specs=[pl.BlockSpec((R, 128), lambda i: (i, 0))] * 8,
        out_shape=[chunk] * 8,
    )(sc[0], sc[1], sc[2], sc[3], deg, b3_row, Wa, Wb)
    return outs[:4], outs[4:]


def _tc_dec_body(g0, g1, g2, g3, w2_ref, bp2_ref, w3_ref, bp3_ref, out_ref):
    g = jnp.concatenate([g0[...], g1[...], g2[...], g3[...]], axis=1)
    o = jnp.dot(g, w2_ref[...], preferred_element_type=jnp.float32)
    o = jnp.maximum(o + bp2_ref[...], 0.0)
    out_ref[...] = jnp.dot(o, w3_ref[...],
                           preferred_element_type=jnp.float32) + bp3_ref[...]


@functools.partial(jax.jit)
def _tc_dec(gc, Wp2, bp2_row, Wp3b, bp3_row):
    grid = ELP // R2
    return pl.pallas_call(
        _tc_dec_body,
        grid=(grid,),
        in_specs=[pl.BlockSpec((R2, 128), lambda i: (i, 0))] * 4 + [
            pl.BlockSpec((H, H // 2), lambda i: (0, 0)),
            pl.BlockSpec((1, H // 2), lambda i: (0, 0)),
            pl.BlockSpec((H // 2, 128), lambda i: (0, 0)),
            pl.BlockSpec((1, 128), lambda i: (0, 0)),
        ],
        out_specs=pl.BlockSpec((R2, 128), lambda i: (i, 0)),
        out_shape=jax.ShapeDtypeStruct((ELP, 128), jnp.float32),
    )(gc[0], gc[1], gc[2], gc[3], Wp2, bp2_row, Wp3b, bp3_row)


def kernel(x, edge_index, edge_label_index, W1, b1, W2, b2, W3, b3,
           g1, be1, g2, be2, Wp1, bp1, Wp2, bp2, Wp3, bp3):
    pad_e = jnp.full((EP - E,), N, jnp.int32)
    src_p = jnp.concatenate([edge_index[0], pad_e])
    dst_p = jnp.concatenate([edge_index[1], pad_e])
    pad_l = jnp.zeros((ELP - EL,), jnp.int32)
    a_p = jnp.concatenate([edge_label_index[0], pad_l])
    b_p = jnp.concatenate([edge_label_index[1], pad_l])
    x_pad = jnp.pad(x, ((0, NP - N), (0, 0)))
    ones16 = jnp.ones((NP, 16), jnp.float32)

    deg = _sc_deg(dst_p, ones16)

    t1 = _tc_proj(x_pad, W1, deg)
    s1 = _sc_seg(t1, src_p, dst_p)
    st1 = _tc_stats(s1, deg, b1.reshape(1, H))
    t2 = _tc_layer(s1, deg, st1, b1.reshape(1, H), g1.reshape(1, H),
                   be1.reshape(1, H), W2)
    s2 = _sc_seg(t2, src_p, dst_p)
    st2 = _tc_stats(s2, deg, b2.reshape(1, H))
    t3 = _tc_layer(s2, deg, st2, b2.reshape(1, H), g2.reshape(1, H),
                   be2.reshape(1, H), W3)
    s3 = _sc_seg(t3, src_p, dst_p)

    p1c, p2c = _tc_zproj(s3, deg, b3.reshape(1, H), Wp1[:H], Wp1[H:])
    gc = _sc_pair(p1c, p2c, a_p, b_p, bp1)
    dec = _tc_dec(gc, Wp2, bp2.reshape(1, H // 2),
                  jnp.broadcast_to(Wp3, (H // 2, 128)),
                  jnp.broadcast_to(bp3.reshape(1, 1), (1, 128)))
    return dec[:EL, 0]


# speedup vs baseline: 4.2232x; 1.7001x over previous
import functools

import jax
import jax.numpy as jnp
from jax import lax
from jax.experimental import pallas as pl
from jax.experimental.pallas import tpu as pltpu
from jax.experimental.pallas import tpu_sc as plsc

N = 10000
NP = 10240
E = 160000
EP = 163840
EL = 160000
ELP = 163840
DIN = 256
H = 512
NC = 2
NS = 16
EB = 128
R = 1024
R2 = 2048

@functools.lru_cache(maxsize=None)
def _get_mesh():
    return plsc.VectorSubcoreMesh(
        core_axis_name="c", subcore_axis_name="s",
        num_cores=NC, num_subcores=NS)


def _sc_deg_body(dst_hbm, ones_hbm, out_hbm, idx_v, ones_v, acc, sem):
    cid = lax.axis_index("c")
    sid = lax.axis_index("s")

    @pl.when(cid == 0)
    def _():
        slab = NP // NS
        r0 = sid * slab
        pltpu.sync_copy(ones_hbm.at[pl.ds(r0, slab)], acc.at[pl.ds(r0, slab)])
        pltpu.sync_copy(ones_hbm.at[pl.ds(0, EB)], ones_v)
        plsc.subcore_barrier()
        epw = EP // NS
        nblk = epw // EB

        def blk(j, carry):
            off = sid * epw + j * EB
            pltpu.sync_copy(dst_hbm.at[pl.ds(off, EB)], idx_v)
            pltpu.sync_copy(ones_v, acc.at[idx_v], add=True)
            return carry

        lax.fori_loop(0, nblk, blk, 0)
        plsc.subcore_barrier()
        pltpu.sync_copy(acc.at[pl.ds(r0, slab)], out_hbm.at[pl.ds(r0, slab)])


@functools.partial(jax.jit)
def _sc_deg(dst_p, ones16):
    return pl.kernel(
        _sc_deg_body,
        out_type=jax.ShapeDtypeStruct((NP, 16), jnp.float32),
        mesh=_get_mesh(),
        scratch_types=[
            pltpu.VMEM((EB,), jnp.int32),
            pltpu.VMEM((EB, 16), jnp.float32),
            pltpu.VMEM_SHARED((NP, 16), jnp.float32),
            pltpu.SemaphoreType.DMA,
        ],
    )(dst_p, ones16)


def _sc_seg_body(t0, t1, t2, t3, src_hbm, dst_hbm, o0, o1, o2, o3,
                 is_all, idx_d0, idx_d1, rbuf0, rbuf1, acc,
                 sem0, sem1, semd0, semd1):
    cid = lax.axis_index("c")
    sid = lax.axis_index("s")
    slab = NP // NS
    r0 = sid * slab
    epw = EP // NS
    nblk = epw // EB
    pltpu.sync_copy(src_hbm.at[pl.ds(sid * epw, epw)], is_all)
    bufs = (rbuf0, rbuf1)
    idxd = (idx_d0, idx_d1)
    sems = (sem0, sem1)
    semd = (semd0, semd1)

    def gather(t_hbm, j, b):
        pltpu.async_copy(
            t_hbm.at[is_all.at[pl.ds(j * EB, EB)]], bufs[b], sems[b])

    def dst_prefetch(j, b):
        pltpu.async_copy(
            dst_hbm.at[pl.ds(sid * epw + j * EB, EB)], idxd[b], semd[b])

    def do_chunk(t_hbm, o_hbm, first, last):
        if first:
            dst_prefetch(0, 0)
            dst_prefetch(1, 1)
        gather(t_hbm, 0, 0)
        gather(t_hbm, 1, 1)
        pltpu.sync_copy(t_hbm.at[pl.ds(r0, slab)], acc.at[pl.ds(r0, slab)])
        plsc.subcore_barrier()

        def step(i, carry):
            for b in range(2):
                j = 2 * i + b
                pltpu.make_async_copy(
                    dst_hbm.at[pl.ds(sid * epw + j * EB, EB)],
                    idxd[b], semd[b]).wait()
                pltpu.make_async_copy(
                    t_hbm.at[is_all.at[pl.ds(j * EB, EB)]],
                    bufs[b], sems[b]).wait()
                pltpu.sync_copy(bufs[b], acc.at[idxd[b]], add=True)

                @pl.when(j + 2 < nblk)
                def _():
                    dst_prefetch(j + 2, b)
                    gather(t_hbm, j + 2, b)

                if not last:
                    @pl.when(j + 2 >= nblk)
                    def _():
                        dst_prefetch(j + 2 - nblk, b)

            return carry

        lax.fori_loop(0, nblk // 2, step, 0)
        plsc.subcore_barrier()
        pltpu.sync_copy(acc.at[pl.ds(r0, slab)], o_hbm.at[pl.ds(r0, slab)])
        plsc.subcore_barrier()

    @pl.when(cid == 0)
    def _():
        do_chunk(t0, o0, True, False)
        do_chunk(t1, o1, False, True)

    @pl.when(cid == 1)
    def _():
        do_chunk(t2, o2, True, False)
        do_chunk(t3, o3, False, True)


@functools.partial(jax.jit)
def _sc_seg(tc, src_p, dst_p):
    chunk = jax.ShapeDtypeStruct((NP, 128), jnp.float32)
    epw = EP // NS
    outs = pl.kernel(
        _sc_seg_body,
        out_type=(chunk,) * 4,
        mesh=_get_mesh(),
        scratch_types=[
            pltpu.VMEM((epw,), jnp.int32),
            pltpu.VMEM((EB,), jnp.int32),
            pltpu.VMEM((EB,), jnp.int32),
            pltpu.VMEM((EB, 128), jnp.float32),
            pltpu.VMEM((EB, 128), jnp.float32),
            pltpu.VMEM_SHARED((NP, 128), jnp.float32),
            pltpu.SemaphoreType.DMA,
            pltpu.SemaphoreType.DMA,
            pltpu.SemaphoreType.DMA,
            pltpu.SemaphoreType.DMA,
        ],
    )(tc[0], tc[1], tc[2], tc[3], src_p, dst_p)
    return outs


EB2 = 40


def _sc_pair_body(p1_hbm, p2_hbm, a_hbm, b_hbm, ga_hbm, gb_hbm,
                  ia_all, ib_all, ra0, ra1, rb0, rb1, sa0, sa1, sb0, sb1):
    cid = lax.axis_index("c")
    sid = lax.axis_index("s")
    wid = sid * NC + cid
    rpw = ELP // (NC * NS)
    nblk = rpw // EB2
    base = wid * rpw
    pltpu.sync_copy(a_hbm.at[pl.ds(base, rpw)], ia_all)
    pltpu.sync_copy(b_hbm.at[pl.ds(base, rpw)], ib_all)
    ras = (ra0, ra1)
    rbs = (rb0, rb1)
    sas = (sa0, sa1)
    sbs = (sb0, sb1)

    def gathers(j, b):
        pltpu.async_copy(
            p1_hbm.at[ia_all.at[pl.ds(j * EB2, EB2)]], ras[b], sas[b])
        pltpu.async_copy(
            p2_hbm.at[ib_all.at[pl.ds(j * EB2, EB2)]], rbs[b], sbs[b])

    gathers(0, 0)
    gathers(1, 1)

    def step(i, carry):
        for b in range(2):
            j = 2 * i + b
            pltpu.make_async_copy(
                p1_hbm.at[ia_all.at[pl.ds(j * EB2, EB2)]], ras[b],
                sas[b]).wait()
            pltpu.make_async_copy(
                p2_hbm.at[ib_all.at[pl.ds(j * EB2, EB2)]], rbs[b],
                sbs[b]).wait()
            roff = base + j * EB2
            pltpu.sync_copy(ras[b], ga_hbm.at[pl.ds(roff, EB2)])
            pltpu.sync_copy(rbs[b], gb_hbm.at[pl.ds(roff, EB2)])

            @pl.when(j + 2 < nblk)
            def _():
                gathers(j + 2, b)

        return carry

    lax.fori_loop(0, nblk // 2, step, 0)


@functools.partial(jax.jit)
def _sc_pair(p1, p2, a_p, b_p):
    out = jax.ShapeDtypeStruct((ELP, H), jnp.float32)
    rpw = ELP // (NC * NS)
    return pl.kernel(
        _sc_pair_body,
        out_type=(out, out),
        mesh=_get_mesh(),
        scratch_types=[
            pltpu.VMEM((rpw,), jnp.int32),
            pltpu.VMEM((rpw,), jnp.int32),
            pltpu.VMEM((EB2, H), jnp.float32),
            pltpu.VMEM((EB2, H), jnp.float32),
            pltpu.VMEM((EB2, H), jnp.float32),
            pltpu.VMEM((EB2, H), jnp.float32),
            pltpu.SemaphoreType.DMA,
            pltpu.SemaphoreType.DMA,
            pltpu.SemaphoreType.DMA,
            pltpu.SemaphoreType.DMA,
        ],
    )(p1, p2, a_p, b_p)


def _dinv_block(deg_blk, i, rows):
    d = deg_blk[:, 0:1]
    ri = lax.broadcasted_iota(jnp.int32, (rows, 1), 0) + i * rows
    return jnp.where(ri < N, lax.rsqrt(d), 0.0)


def _tc_proj_body(x_ref, w_ref, deg_ref, o0, o1, o2, o3):
    i = pl.program_id(0)
    m = jnp.dot(x_ref[...], w_ref[...], preferred_element_type=jnp.float32)
    t = m * _dinv_block(deg_ref[...], i, R)
    o0[...] = t[:, 0:128]
    o1[...] = t[:, 128:256]
    o2[...] = t[:, 256:384]
    o3[...] = t[:, 384:512]


@functools.partial(jax.jit)
def _tc_proj(x_pad, W1, deg):
    chunk = jax.ShapeDtypeStruct((NP, 128), jnp.float32)
    grid = NP // R
    return pl.pallas_call(
        _tc_proj_body,
        grid=(grid,),
        in_specs=[
            pl.BlockSpec((R, DIN), lambda i: (i, 0)),
            pl.BlockSpec((DIN, H), lambda i: (0, 0)),
            pl.BlockSpec((R, 16), lambda i: (i, 0)),
        ],
        out_specs=[pl.BlockSpec((R, 128), lambda i: (i, 0))] * 4,
        out_shape=[chunk] * 4,
    )(x_pad, W1, deg)


def _y_block(s_refs, deg_ref, b_ref, i):
    s = jnp.concatenate([r[...] for r in s_refs], axis=1)
    dinv = _dinv_block(deg_ref[...], i, R)
    return s * dinv + b_ref[...], dinv


def _tc_stats_body(s0, s1, s2, s3, deg_ref, b_ref, out_ref):
    i = pl.program_id(0)
    y, _ = _y_block((s0, s1, s2, s3), deg_ref, b_ref, i)
    ri = lax.broadcasted_iota(jnp.int32, (R, 1), 0) + i * R
    ym = jnp.where(ri < N, y, 0.0)
    su = jnp.sum(ym, axis=0, keepdims=True)
    sq = jnp.sum(ym * ym, axis=0, keepdims=True)
    upd = jnp.concatenate([su, sq, jnp.zeros((6, H), jnp.float32)], axis=0)

    @pl.when(i == 0)
    def _():
        out_ref[...] = jnp.zeros((8, H), jnp.float32)

    out_ref[...] += upd


@functools.partial(jax.jit)
def _tc_stats(sc, deg, b_row):
    grid = NP // R
    return pl.pallas_call(
        _tc_stats_body,
        grid=(grid,),
        in_specs=[pl.BlockSpec((R, 128), lambda i: (i, 0))] * 4 + [
            pl.BlockSpec((R, 16), lambda i: (i, 0)),
            pl.BlockSpec((1, H), lambda i: (0, 0)),
        ],
        out_specs=pl.BlockSpec((8, H), lambda i: (0, 0)),
        out_shape=jax.ShapeDtypeStruct((8, H), jnp.float32),
    )(sc[0], sc[1], sc[2], sc[3], deg, b_row)


def _tc_layer_body(s0, s1, s2, s3, deg_ref, st_ref, b_ref, g_ref, be_ref,
                   w_ref, o0, o1, o2, o3):
    i = pl.program_id(0)
    y, dinv = _y_block((s0, s1, s2, s3), deg_ref, b_ref, i)
    st = st_ref[...]
    mu = st[0:1, :] * (1.0 / N)
    var = st[1:2, :] * (1.0 / N) - mu * mu
    hn = (y - mu) * lax.rsqrt(var + 1e-5) * g_ref[...] + be_ref[...]
    h = jnp.maximum(hn, 0.0)
    t = jnp.dot(h, w_ref[...], preferred_element_type=jnp.float32) * dinv
    o0[...] = t[:, 0:128]
    o1[...] = t[:, 128:256]
    o2[...] = t[:, 256:384]
    o3[...] = t[:, 384:512]


@functools.partial(jax.jit)
def _tc_layer(sc, deg, st, b_row, g_row, be_row, Wn):
    chunk = jax.ShapeDtypeStruct((NP, 128), jnp.float32)
    grid = NP // R
    return pl.pallas_call(
        _tc_layer_body,
        grid=(grid,),
        in_specs=[pl.BlockSpec((R, 128), lambda i: (i, 0))] * 4 + [
            pl.BlockSpec((R, 16), lambda i: (i, 0)),
            pl.BlockSpec((8, H), lambda i: (0, 0)),
            pl.BlockSpec((1, H), lambda i: (0, 0)),
            pl.BlockSpec((1, H), lambda i: (0, 0)),
            pl.BlockSpec((1, H), lambda i: (0, 0)),
            pl.BlockSpec((H, H), lambda i: (0, 0)),
        ],
        out_specs=[pl.BlockSpec((R, 128), lambda i: (i, 0))] * 4,
        out_shape=[chunk] * 4,
    )(sc[0], sc[1], sc[2], sc[3], deg, st, b_row, g_row, be_row, Wn)


def _tc_zproj_body(s0, s1, s2, s3, deg_ref, b_ref, wa_ref, wb_ref, o1, o2):
    i = pl.program_id(0)
    z, _ = _y_block((s0, s1, s2, s3), deg_ref, b_ref, i)
    o1[...] = jnp.dot(z, wa_ref[...], preferred_element_type=jnp.float32)
    o2[...] = jnp.dot(z, wb_ref[...], preferred_element_type=jnp.float32)


@functools.partial(jax.jit)
def _tc_zproj(sc, deg, b3_row, Wa, Wb):
    full = jax.ShapeDtypeStruct((NP, H), jnp.float32)
    grid = NP // R
    outs = pl.pallas_call(
        _tc_zproj_body,
        grid=(grid,),
        in_specs=[pl.BlockSpec((R, 128), lambda i: (i, 0))] * 4 + [
            pl.BlockSpec((R, 16), lambda i: (i, 0)),
            pl.BlockSpec((1, H), lambda i: (0, 0)),
            pl.BlockSpec((H, H), lambda i: (0, 0)),
            pl.BlockSpec((H, H), lambda i: (0, 0)),
        ],
        out_specs=[pl.BlockSpec((R, H), lambda i: (i, 0))] * 2,
        out_shape=[full] * 2,
    )(sc[0], sc[1], sc[2], sc[3], deg, b3_row, Wa, Wb)
    return outs


def _tc_dec_body(ga_ref, gb_ref, bp1_ref, w2_ref, bp2_ref, w3_ref, bp3_ref,
                 out_ref):
    g = jnp.maximum(ga_ref[...] + gb_ref[...] + bp1_ref[...], 0.0)
    o = jnp.dot(g, w2_ref[...], preferred_element_type=jnp.float32)
    o = jnp.maximum(o + bp2_ref[...], 0.0)
    out_ref[...] = jnp.dot(o, w3_ref[...],
                           preferred_element_type=jnp.float32) + bp3_ref[...]


@functools.partial(jax.jit)
def _tc_dec(ga, gb, bp1_row, Wp2, bp2_row, Wp3b, bp3_row):
    grid = ELP // R2
    return pl.pallas_call(
        _tc_dec_body,
        grid=(grid,),
        in_specs=[
            pl.BlockSpec((R2, H), lambda i: (i, 0)),
            pl.BlockSpec((R2, H), lambda i: (i, 0)),
            pl.BlockSpec((1, H), lambda i: (0, 0)),
            pl.BlockSpec((H, H // 2), lambda i: (0, 0)),
            pl.BlockSpec((1, H // 2), lambda i: (0, 0)),
            pl.BlockSpec((H // 2, 128), lambda i: (0, 0)),
            pl.BlockSpec((1, 128), lambda i: (0, 0)),
        ],
        out_specs=pl.BlockSpec((R2, 128), lambda i: (i, 0)),
        out_shape=jax.ShapeDtypeStruct((ELP, 128), jnp.float32),
    )(ga, gb, bp1_row, Wp2, bp2_row, Wp3b, bp3_row)


def kernel(x, edge_index, edge_label_index, W1, b1, W2, b2, W3, b3,
           g1, be1, g2, be2, Wp1, bp1, Wp2, bp2, Wp3, bp3):
    pad_e = jnp.full((EP - E,), N, jnp.int32)
    src_p = jnp.concatenate([edge_index[0], pad_e])
    dst_p = jnp.concatenate([edge_index[1], pad_e])
    pad_l = jnp.zeros((ELP - EL,), jnp.int32)
    a_p = jnp.concatenate([edge_label_index[0], pad_l])
    b_p = jnp.concatenate([edge_label_index[1], pad_l])
    x_pad = jnp.pad(x, ((0, NP - N), (0, 0)))
    ones16 = jnp.ones((NP, 16), jnp.float32)

    deg = _sc_deg(dst_p, ones16)

    t1 = _tc_proj(x_pad, W1, deg)
    s1 = _sc_seg(t1, src_p, dst_p)
    st1 = _tc_stats(s1, deg, b1.reshape(1, H))
    t2 = _tc_layer(s1, deg, st1, b1.reshape(1, H), g1.reshape(1, H),
                   be1.reshape(1, H), W2)
    s2 = _sc_seg(t2, src_p, dst_p)
    st2 = _tc_stats(s2, deg, b2.reshape(1, H))
    t3 = _tc_layer(s2, deg, st2, b2.reshape(1, H), g2.reshape(1, H),
                   be2.reshape(1, H), W3)
    s3 = _sc_seg(t3, src_p, dst_p)

    p1, p2 = _tc_zproj(s3, deg, b3.reshape(1, H), Wp1[:H], Wp1[H:])
    ga, gb = _sc_pair(p1, p2, a_p, b_p)
    dec = _tc_dec(ga, gb, bp1.reshape(1, H), Wp2, bp2.reshape(1, H // 2),
                  jnp.broadcast_to(Wp3, (H // 2, 128)),
                  jnp.broadcast_to(bp3.reshape(1, 1), (1, 128)))
    return dec[:EL, 0]
